# baseline (speedup 1.0000x reference)
import functools

import jax
import jax.numpy as jnp
from jax import lax
from jax.experimental import pallas as pl
from jax.experimental.pallas import tpu as pltpu
from jax.experimental.pallas import tpu_sc as plsc

_NUM_CORES = 2
_NUM_SUBCORES = 16
_NW = _NUM_CORES * _NUM_SUBCORES


@jax.jit
def _emb_lookup(x, W):
    B, T = x.shape
    _, D = W.shape
    rows_per_w = B // _NW
    mesh = plsc.VectorSubcoreMesh(core_axis_name="c", subcore_axis_name="s")

    @functools.partial(
        pl.kernel,
        out_type=jax.ShapeDtypeStruct((B, T, D), jnp.float32),
        mesh=mesh,
        compiler_params=pltpu.CompilerParams(use_tc_tiling_on_sc=False),
        scratch_types=[
            pltpu.VMEM((rows_per_w, T), jnp.int32),
            pltpu.VMEM((T, D), jnp.float32),
            pltpu.VMEM((T, D), jnp.float32),
            pltpu.SemaphoreType.DMA,
            pltpu.SemaphoreType.DMA,
            pltpu.SemaphoreType.DMA,
            pltpu.SemaphoreType.DMA,
        ],
    )
    def k(x_hbm, table_hbm, out_hbm, idx_v, buf0, buf1, gs0, gs1, ss0, ss1):
        wid = lax.axis_index("s") * _NUM_CORES + lax.axis_index("c")
        b0 = wid * rows_per_w
        pltpu.sync_copy(x_hbm.at[pl.ds(b0, rows_per_w)], idx_v)

        def g_start(j, buf, sem):
            pltpu.async_copy(table_hbm.at[idx_v.at[j, :]], buf, sem)

        def g_wait(buf, sem):
            pltpu.make_async_copy(table_hbm.at[idx_v.at[0, :]], buf, sem).wait()

        def s_start(j, buf, sem):
            pltpu.async_copy(buf, out_hbm.at[b0 + j], sem)

        def s_wait(buf, sem):
            pltpu.make_async_copy(buf, out_hbm.at[b0], sem).wait()

        g_start(0, buf0, gs0)
        n2 = rows_per_w // 2

        @pl.loop(0, n2)
        def _(g):
            j0 = 2 * g

            @pl.when(g > 0)
            def _():
                s_wait(buf1, ss1)

            g_start(j0 + 1, buf1, gs1)
            g_wait(buf0, gs0)
            s_start(j0, buf0, ss0)

            @pl.when(g < n2 - 1)
            def _():
                s_wait(buf0, ss0)
                g_start(j0 + 2, buf0, gs0)

            g_wait(buf1, gs1)
            s_start(j0 + 1, buf1, ss1)

        s_wait(buf0, ss0)
        s_wait(buf1, ss1)

    return k(x, W)


def kernel(x, W):
    return _emb_lookup(x, W)


# speedup vs baseline: 1.2211x; 1.2211x over previous
import functools

import jax
import jax.numpy as jnp
from jax import lax
from jax.experimental import pallas as pl
from jax.experimental.pallas import tpu as pltpu
from jax.experimental.pallas import tpu_sc as plsc

_NUM_CORES = 2
_NUM_SUBCORES = 16
_NW = _NUM_CORES * _NUM_SUBCORES


@functools.partial(jax.jit, static_argnums=(2,))
def _emb_gather(Wp, idx, B):
    DP = Wp.shape[1]
    b_per_w = B // _NW
    CH = 256
    n_chunks = b_per_w // CH
    assert n_chunks % 2 == 0
    mesh = plsc.VectorSubcoreMesh(core_axis_name="c", subcore_axis_name="s")

    @functools.partial(
        pl.kernel,
        out_type=jax.ShapeDtypeStruct((B, DP), jnp.float32),
        mesh=mesh,
        compiler_params=pltpu.CompilerParams(use_tc_tiling_on_sc=True),
        scratch_types=[
            pltpu.VMEM((b_per_w,), jnp.int32),
            pltpu.VMEM((CH, DP), jnp.float32),
            pltpu.VMEM((CH, DP), jnp.float32),
            pltpu.SemaphoreType.DMA,
            pltpu.SemaphoreType.DMA,
            pltpu.SemaphoreType.DMA,
            pltpu.SemaphoreType.DMA,
        ],
    )
    def k(table_hbm, idx_hbm, out_hbm, idx_v, buf0, buf1, gs0, gs1, ss0, ss1):
        wid = lax.axis_index("s") * _NUM_CORES + lax.axis_index("c")
        base = wid * b_per_w
        pltpu.sync_copy(idx_hbm.at[pl.ds(base, b_per_w)], idx_v)

        def g_start(i, buf, sem):
            pltpu.async_copy(table_hbm.at[idx_v.at[pl.ds(i * CH, CH)]], buf, sem)

        def g_wait(buf, sem):
            pltpu.make_async_copy(
                table_hbm.at[idx_v.at[pl.ds(0, CH)]], buf, sem
            ).wait()

        def s_start(i, buf, sem):
            pltpu.async_copy(buf, out_hbm.at[pl.ds(base + i * CH, CH)], sem)

        def s_wait(buf, sem):
            pltpu.make_async_copy(buf, out_hbm.at[pl.ds(base, CH)], sem).wait()

        g_start(0, buf0, gs0)
        n2 = n_chunks // 2

        @pl.loop(0, n2)
        def _(g):
            i0 = 2 * g

            @pl.when(g > 0)
            def _():
                s_wait(buf1, ss1)

            g_start(i0 + 1, buf1, gs1)
            g_wait(buf0, gs0)
            s_start(i0, buf0, ss0)

            @pl.when(g < n2 - 1)
            def _():
                s_wait(buf0, ss0)
                g_start(i0 + 2, buf0, gs0)

            g_wait(buf1, gs1)
            s_start(i0 + 1, buf1, ss1)

        s_wait(buf0, ss0)
        s_wait(buf1, ss1)

    return k(Wp, idx)


def kernel(x, W):
    B, T = x.shape
    V, D = W.shape
    Wp = jnp.pad(W, ((0, 0), (0, 128 - D)))
    idx = x.reshape(-1)
    outp = _emb_gather(Wp, idx, B * T)
    return outp.reshape(B, T, 128)[:, :, :D]
